# baseline (speedup 1.0000x reference)
import functools

import jax
import jax.numpy as jnp
from jax import lax
from jax.experimental import pallas as pl
from jax.experimental.pallas import tpu as pltpu
from jax.experimental.pallas import tpu_sc as plsc

_IDX_CHUNK = 128


def _sc_gather(table, idx3d, n_rows, d):
    info = plsc.get_sparse_core_info()
    nw = info.num_cores * info.num_subcores
    rows_per_w = n_rows // nw
    chunks_per_w = rows_per_w // _IDX_CHUNK

    mesh = plsc.VectorSubcoreMesh(core_axis_name="c", subcore_axis_name="s")

    @functools.partial(
        pl.kernel,
        mesh=mesh,
        compiler_params=pltpu.CompilerParams(use_tc_tiling_on_sc=False),
        out_type=jax.ShapeDtypeStruct((n_rows, d), jnp.float32),
        scratch_types=[
            pltpu.VMEM((chunks_per_w, _IDX_CHUNK), jnp.int32),
            pltpu.VMEM((rows_per_w, d), jnp.float32),
            pltpu.SemaphoreType.DMA,
        ],
    )
    def gather_kernel(table_hbm, idx_hbm, out_hbm, idx_v, rows_v, sem):
        wid = lax.axis_index("s") * info.num_cores + lax.axis_index("c")
        pltpu.sync_copy(idx_hbm.at[wid], idx_v)
        copies = []
        for j in range(chunks_per_w):
            copies.append(
                pltpu.async_copy(
                    table_hbm.at[idx_v.at[j]],
                    rows_v.at[pl.ds(j * _IDX_CHUNK, _IDX_CHUNK)],
                    sem,
                )
            )
        for c in copies:
            c.wait()
        pltpu.sync_copy(rows_v, out_hbm.at[pl.ds(wid * rows_per_w, rows_per_w)])

    return gather_kernel(table, idx3d)


def _mlp(h, W1, b1, W2, b2, W3, b3, tile_v):
    batch, feat = h.shape
    hid1 = W1.shape[1]
    hid2 = W2.shape[1]
    vocab = W3.shape[1]
    n_tiles = pl.cdiv(vocab, tile_v)

    def body(h_ref, W1_ref, b1_ref, W2_ref, b2_ref, W3_ref, b3_ref,
             out_ref, h2_ref):
        @pl.when(pl.program_id(0) == 0)
        def _compute_hidden():
            a = jnp.dot(h_ref[...], W1_ref[...],
                        preferred_element_type=jnp.float32) + b1_ref[...]
            a = jnp.maximum(a, 0.0)
            b = jnp.dot(a, W2_ref[...],
                        preferred_element_type=jnp.float32) + b2_ref[...]
            h2_ref[...] = jnp.maximum(b, 0.0)

        out_ref[...] = jnp.dot(h2_ref[...], W3_ref[...],
                               preferred_element_type=jnp.float32) + b3_ref[...]

    return pl.pallas_call(
        body,
        grid=(n_tiles,),
        in_specs=[
            pl.BlockSpec((batch, feat), lambda j: (0, 0)),
            pl.BlockSpec((feat, hid1), lambda j: (0, 0)),
            pl.BlockSpec((1, hid1), lambda j: (0, 0)),
            pl.BlockSpec((hid1, hid2), lambda j: (0, 0)),
            pl.BlockSpec((1, hid2), lambda j: (0, 0)),
            pl.BlockSpec((hid2, tile_v), lambda j: (0, j)),
            pl.BlockSpec((1, tile_v), lambda j: (0, j)),
        ],
        out_specs=pl.BlockSpec((batch, tile_v), lambda j: (0, j)),
        out_shape=jax.ShapeDtypeStruct((batch, vocab), jnp.float32),
        scratch_shapes=[pltpu.VMEM((batch, hid2), jnp.float32)],
        compiler_params=pltpu.CompilerParams(
            dimension_semantics=("arbitrary",),
        ),
    )(h, W1, b1, W2, b2, W3, b3)


def kernel(x, table, W1, b1, W2, b2, W3, b3):
    batch, k = x.shape
    d = table.shape[1]
    idx = x.astype(jnp.int32).reshape(32, -1, _IDX_CHUNK)
    emb = _sc_gather(table, idx, batch * k, d)
    h = emb.reshape(batch, k * d)
    return _mlp(h, W1, b1.reshape(1, -1), W2, b2.reshape(1, -1),
                W3, b3.reshape(1, -1), tile_v=2048)


# speedup vs baseline: 2.0369x; 2.0369x over previous
import functools

import jax
import jax.numpy as jnp
from jax import lax
from jax.experimental import pallas as pl
from jax.experimental.pallas import tpu as pltpu
from jax.experimental.pallas import tpu_sc as plsc

_IDX_CHUNK = 128


def _sc_gather(table, idx3d, n_rows, d):
    info = plsc.get_sparse_core_info()
    nw = info.num_cores * info.num_subcores
    rows_per_w = n_rows // nw
    chunks_per_w = rows_per_w // _IDX_CHUNK

    mesh = plsc.VectorSubcoreMesh(core_axis_name="c", subcore_axis_name="s")

    @functools.partial(
        pl.kernel,
        mesh=mesh,
        compiler_params=pltpu.CompilerParams(use_tc_tiling_on_sc=False),
        out_type=jax.ShapeDtypeStruct((n_rows, d), jnp.float32),
        scratch_types=[
            pltpu.VMEM((chunks_per_w, _IDX_CHUNK), jnp.int32),
            pltpu.VMEM((rows_per_w, d), jnp.float32),
            pltpu.SemaphoreType.DMA,
        ],
    )
    def gather_kernel(table_hbm, idx_hbm, out_hbm, idx_v, rows_v, sem):
        wid = lax.axis_index("s") * info.num_cores + lax.axis_index("c")
        pltpu.sync_copy(idx_hbm.at[wid], idx_v)
        copies = []
        for j in range(chunks_per_w):
            copies.append(
                pltpu.async_copy(
                    table_hbm.at[idx_v.at[j]],
                    rows_v.at[pl.ds(j * _IDX_CHUNK, _IDX_CHUNK)],
                    sem,
                )
            )
        for c in copies:
            c.wait()
        pltpu.sync_copy(rows_v, out_hbm.at[pl.ds(wid * rows_per_w, rows_per_w)])

    return gather_kernel(table, idx3d)


def _mlp(h, W1, b1, W2, b2, W3, b3, tile_v):
    batch, feat = h.shape
    hid1 = W1.shape[1]
    hid2 = W2.shape[1]
    vocab = W3.shape[1]
    n_tiles = pl.cdiv(vocab, tile_v)

    def body(h_ref, W1_ref, b1_ref, W2_ref, b2_ref, W3_ref, b3_ref,
             out_ref, h2_ref):
        @pl.when(pl.program_id(0) == 0)
        def _compute_hidden():
            a = jnp.dot(h_ref[...], W1_ref[...],
                        preferred_element_type=jnp.float32) + b1_ref[...]
            a = jnp.maximum(a, 0.0)
            b = jnp.dot(a, W2_ref[...],
                        preferred_element_type=jnp.float32) + b2_ref[...]
            h2_ref[...] = jnp.maximum(b, 0.0)

        dot_t = lax.dot_general(
            W3_ref[...], h2_ref[...],
            dimension_numbers=(((0,), (1,)), ((), ())),
            preferred_element_type=jnp.float32,
        )
        out_ref[...] = dot_t + b3_ref[...]

    return pl.pallas_call(
        body,
        grid=(n_tiles,),
        in_specs=[
            pl.BlockSpec((batch, feat), lambda j: (0, 0)),
            pl.BlockSpec((feat, hid1), lambda j: (0, 0)),
            pl.BlockSpec((1, hid1), lambda j: (0, 0)),
            pl.BlockSpec((hid1, hid2), lambda j: (0, 0)),
            pl.BlockSpec((1, hid2), lambda j: (0, 0)),
            pl.BlockSpec((hid2, tile_v), lambda j: (0, j)),
            pl.BlockSpec((tile_v, 1), lambda j: (j, 0)),
        ],
        out_specs=pl.BlockSpec((tile_v, batch), lambda j: (j, 0)),
        out_shape=jax.ShapeDtypeStruct((vocab, batch), jnp.float32),
        scratch_shapes=[pltpu.VMEM((batch, hid2), jnp.float32)],
        compiler_params=pltpu.CompilerParams(
            dimension_semantics=("arbitrary",),
        ),
    )(h, W1, b1, W2, b2, W3, b3)


def kernel(x, table, W1, b1, W2, b2, W3, b3):
    batch, k = x.shape
    d = table.shape[1]
    idx = x.astype(jnp.int32).reshape(32, -1, _IDX_CHUNK)
    emb = _sc_gather(table, idx, batch * k, d)
    h = emb.reshape(batch, k * d)
    out_t = _mlp(h, W1, b1.reshape(1, -1), W2, b2.reshape(1, -1),
                 W3, b3.reshape(-1, 1), tile_v=2048)
    return out_t.T


# speedup vs baseline: 2.5091x; 1.2318x over previous
import functools

import jax
import jax.numpy as jnp
from jax import lax
from jax.experimental import pallas as pl
from jax.experimental.pallas import tpu as pltpu
from jax.experimental.pallas import tpu_sc as plsc

_IDX_CHUNK = 128


def _sc_gather(table, xt3d, n_rows, d):
    info = plsc.get_sparse_core_info()
    nw = info.num_cores * info.num_subcores
    rows_per_w = n_rows // nw
    chunks_per_w = rows_per_w // _IDX_CHUNK

    mesh = plsc.VectorSubcoreMesh(core_axis_name="c", subcore_axis_name="s")

    @functools.partial(
        pl.kernel,
        mesh=mesh,
        compiler_params=pltpu.CompilerParams(use_tc_tiling_on_sc=False),
        out_type=jax.ShapeDtypeStruct((n_rows, d), jnp.float32),
        scratch_types=[
            pltpu.VMEM((chunks_per_w, _IDX_CHUNK), jnp.int32),
            pltpu.VMEM((rows_per_w, d), jnp.float32),
            pltpu.SemaphoreType.DMA,
        ],
    )
    def gather_kernel(table_hbm, xt_hbm, out_hbm, idx_v, rows_v, sem):
        wid = lax.axis_index("s") * info.num_cores + lax.axis_index("c")
        pltpu.sync_copy(xt_hbm.at[wid], idx_v)
        copies = []
        for j in range(chunks_per_w):
            copies.append(
                pltpu.async_copy(
                    table_hbm.at[idx_v.at[j]],
                    rows_v.at[pl.ds(j * _IDX_CHUNK, _IDX_CHUNK)],
                    sem,
                )
            )
        for c in copies:
            c.wait()
        pltpu.sync_copy(rows_v, out_hbm.at[pl.ds(wid * rows_per_w, rows_per_w)])

    return gather_kernel(table, xt3d)


def _mlp(emb3, W13, b1, W2, b2, W3, b3, tile_v):
    k_per_b, batch, d = emb3.shape
    hid1 = W13.shape[2]
    hid2 = W2.shape[1]
    vocab = W3.shape[1]
    n_tiles = pl.cdiv(vocab, tile_v)

    def body(emb_ref, W1_ref, b1_ref, W2_ref, b2_ref, W3_ref, b3_ref,
             out_ref, h2_ref):
        @pl.when(pl.program_id(0) == 0)
        def _compute_hidden():
            a = b1_ref[...]
            for k in range(k_per_b):
                a = a + jnp.dot(emb_ref[k], W1_ref[k],
                                preferred_element_type=jnp.float32)
            a = jnp.maximum(a, 0.0)
            b = jnp.dot(a, W2_ref[...],
                        preferred_element_type=jnp.float32) + b2_ref[...]
            h2_ref[...] = jnp.maximum(b, 0.0)

        dot_t = lax.dot_general(
            W3_ref[...], h2_ref[...],
            dimension_numbers=(((0,), (1,)), ((), ())),
            preferred_element_type=jnp.float32,
        )
        bias_t = lax.dot_general(
            b3_ref[...], jnp.ones((1, batch), jnp.float32),
            dimension_numbers=(((0,), (0,)), ((), ())),
            preferred_element_type=jnp.float32,
        )
        out_ref[...] = dot_t + bias_t

    return pl.pallas_call(
        body,
        grid=(n_tiles,),
        in_specs=[
            pl.BlockSpec((k_per_b, batch, d), lambda j: (0, 0, 0)),
            pl.BlockSpec((k_per_b, d, hid1), lambda j: (0, 0, 0)),
            pl.BlockSpec((1, hid1), lambda j: (0, 0)),
            pl.BlockSpec((hid1, hid2), lambda j: (0, 0)),
            pl.BlockSpec((1, hid2), lambda j: (0, 0)),
            pl.BlockSpec((hid2, tile_v), lambda j: (0, j)),
            pl.BlockSpec((1, tile_v), lambda j: (0, j)),
        ],
        out_specs=pl.BlockSpec((tile_v, batch), lambda j: (j, 0)),
        out_shape=jax.ShapeDtypeStruct((vocab, batch), jnp.float32),
        scratch_shapes=[pltpu.VMEM((batch, hid2), jnp.float32)],
        compiler_params=pltpu.CompilerParams(
            dimension_semantics=("arbitrary",),
        ),
    )(emb3, W13, b1, W2, b2, W3, b3)


def kernel(x, table, W1, b1, W2, b2, W3, b3):
    batch, k = x.shape
    d = table.shape[1]
    xt3d = x.astype(jnp.int32).T.reshape(32, -1, _IDX_CHUNK)
    emb = _sc_gather(table, xt3d, batch * k, d)
    emb3 = emb.reshape(k, batch, d)
    W13 = W1.reshape(k, d, W1.shape[1])
    out_t = _mlp(emb3, W13, b1.reshape(1, -1), W2, b2.reshape(1, -1),
                 W3, b3.reshape(1, -1), tile_v=2048)
    return out_t.T


# speedup vs baseline: 2.5168x; 1.0031x over previous
import functools

import jax
import jax.numpy as jnp
from jax import lax
from jax.experimental import pallas as pl
from jax.experimental.pallas import tpu as pltpu
from jax.experimental.pallas import tpu_sc as plsc

_IDX_CHUNK = 128


def _sc_gather(table, xt3d, n_rows, d):
    info = plsc.get_sparse_core_info()
    nw = info.num_cores * info.num_subcores
    rows_per_w = n_rows // nw
    chunks_per_w = rows_per_w // _IDX_CHUNK

    mesh = plsc.VectorSubcoreMesh(core_axis_name="c", subcore_axis_name="s")

    @functools.partial(
        pl.kernel,
        mesh=mesh,
        compiler_params=pltpu.CompilerParams(use_tc_tiling_on_sc=False),
        out_type=jax.ShapeDtypeStruct((n_rows, d), jnp.float32),
        scratch_types=[
            pltpu.VMEM((chunks_per_w, _IDX_CHUNK), jnp.int32),
            pltpu.VMEM((rows_per_w, d), jnp.float32),
            pltpu.SemaphoreType.DMA,
        ],
    )
    def gather_kernel(xt_hbm, table_hbm, out_hbm, idx_v, rows_v, sem):
        wid = lax.axis_index("s") * info.num_cores + lax.axis_index("c")
        pltpu.sync_copy(xt_hbm.at[wid], idx_v)
        copies = []
        for j in range(chunks_per_w):
            copies.append(
                pltpu.async_copy(
                    table_hbm.at[idx_v.at[j]],
                    rows_v.at[pl.ds(j * _IDX_CHUNK, _IDX_CHUNK)],
                    sem,
                )
            )
        for c in copies:
            c.wait()
        pltpu.sync_copy(rows_v, out_hbm.at[pl.ds(wid * rows_per_w, rows_per_w)])

    return gather_kernel(xt3d, table)


def _mlp(emb3, W13, b1, W2, b2, W3, b3, tile_v):
    k_per_b, batch, d = emb3.shape
    hid1 = W13.shape[2]
    hid2 = W2.shape[1]
    vocab = W3.shape[1]
    n_tiles = pl.cdiv(vocab, tile_v)

    def body(emb_ref, W1_ref, b1_ref, W2_ref, b2_ref, W3_ref, b3_ref,
             out_ref, h2_ref):
        @pl.when(pl.program_id(0) == 0)
        def _compute_hidden():
            a = b1_ref[...]
            for k in range(k_per_b):
                a = a + jnp.dot(emb_ref[k], W1_ref[k],
                                preferred_element_type=jnp.float32)
            a = jnp.maximum(a, 0.0)
            b = jnp.dot(a, W2_ref[...],
                        preferred_element_type=jnp.float32) + b2_ref[...]
            h2_ref[...] = jnp.maximum(b, 0.0)

        dot_t = lax.dot_general(
            W3_ref[...], h2_ref[...],
            dimension_numbers=(((0,), (1,)), ((), ())),
            preferred_element_type=jnp.float32,
        )
        bias_t = lax.dot_general(
            b3_ref[...], jnp.ones((1, batch), jnp.float32),
            dimension_numbers=(((0,), (0,)), ((), ())),
            preferred_element_type=jnp.float32,
        )
        out_ref[...] = dot_t + bias_t

    return pl.pallas_call(
        body,
        grid=(n_tiles,),
        in_specs=[
            pl.BlockSpec((k_per_b, batch, d), lambda j: (0, 0, 0)),
            pl.BlockSpec((k_per_b, d, hid1), lambda j: (0, 0, 0)),
            pl.BlockSpec((1, hid1), lambda j: (0, 0)),
            pl.BlockSpec((hid1, hid2), lambda j: (0, 0)),
            pl.BlockSpec((1, hid2), lambda j: (0, 0)),
            pl.BlockSpec((hid2, tile_v), lambda j: (0, j)),
            pl.BlockSpec((1, tile_v), lambda j: (0, j)),
        ],
        out_specs=pl.BlockSpec((tile_v, batch), lambda j: (j, 0)),
        out_shape=jax.ShapeDtypeStruct((vocab, batch), jnp.float32),
        scratch_shapes=[pltpu.VMEM((batch, hid2), jnp.float32)],
        compiler_params=pltpu.CompilerParams(
            dimension_semantics=("arbitrary",),
        ),
    )(emb3, W13, b1, W2, b2, W3, b3)


def kernel(x, table, W1, b1, W2, b2, W3, b3):
    batch, k = x.shape
    d = table.shape[1]
    xt3d = x.astype(jnp.int32).T.reshape(32, -1, _IDX_CHUNK)
    emb = _sc_gather(table, xt3d, batch * k, d)
    emb3 = emb.reshape(k, batch, d)
    W13 = W1.reshape(k, d, W1.shape[1])
    out_t = _mlp(emb3, W13, b1.reshape(1, -1), W2, b2.reshape(1, -1),
                 W3, b3.reshape(1, -1), tile_v=4096)
    return out_t.T
